# baseline (speedup 1.0000x reference)
import functools

import jax
import jax.numpy as jnp
from jax import lax
from jax.experimental import pallas as pl
from jax.experimental.pallas import tpu as pltpu
from jax.experimental.pallas import tpu_sc as plsc

T = 4096
H = 2048
FF = 1408
E = 8
A = 2 * T
BLK = 256
NT = A // BLK + E
P = NT * BLK
NW = 32
CH = 16
NCH = (A // NW) // CH

_ABLK = 512
_NABLK = A // _ABLK
_RBLK = 2048


def _router_body(x_ref, gw_ref, pos_ref, w1_ref, w2_ref, eot_ref, logits):
    i = pl.program_id(0)
    nx = pl.num_programs(0) - 1

    @pl.when(i < nx)
    def _():
        xb = x_ref[...].astype(jnp.bfloat16)
        gw = gw_ref[...].astype(jnp.bfloat16)
        lgb = lax.dot(xb, gw, preferred_element_type=jnp.float32)
        logits[pl.ds(i * _RBLK, _RBLK), :] = lgb

    @pl.when(i == nx)
    def _():
        lgT = jnp.transpose(logits[...])
        row = lax.broadcasted_iota(jnp.int32, (E, T), 0)
        m1 = jnp.max(lgT, axis=0, keepdims=True)
        i1 = jnp.min(jnp.where(lgT == m1, row, E), axis=0, keepdims=True)
        lg2 = jnp.where(row == i1, -jnp.inf, lgT)
        m2 = jnp.max(lg2, axis=0, keepdims=True)
        i2 = jnp.min(jnp.where(lg2 == m2, row, E), axis=0, keepdims=True)
        s2 = jnp.exp(m2 - m1)
        w1 = 1.0 / (1.0 + s2)
        w2 = s2 / (1.0 + s2)
        w1_ref[...] = w1.reshape(T // 128, 128)
        w2_ref[...] = w2.reshape(T // 128, 128)

        row8a = lax.broadcasted_iota(jnp.int32, (E, A), 0)
        e_asn = jnp.concatenate([jnp.broadcast_to(i1, (1, T)),
                                 jnp.broadcast_to(i2, (1, T))], axis=1)
        onehot = (jnp.broadcast_to(e_asn, (E, A)) == row8a
                  ).astype(jnp.float32)

        r = lax.broadcasted_iota(jnp.int32, (_ABLK, _ABLK), 0)
        c = lax.broadcasted_iota(jnp.int32, (_ABLK, _ABLK), 1)
        triu = (r < c).astype(jnp.float32)
        ranks = []
        tots = []
        for b in range(_NABLK):
            ob = onehot[:, b * _ABLK:(b + 1) * _ABLK]
            ranks.append(lax.dot(ob, triu, preferred_element_type=jnp.float32))
            tots.append(jnp.sum(ob, axis=1, keepdims=True))
        tot = jnp.concatenate(tots, axis=1)
        rb = lax.broadcasted_iota(jnp.int32, (_NABLK, _NABLK), 0)
        cb = lax.broadcasted_iota(jnp.int32, (_NABLK, _NABLK), 1)
        triub = (rb < cb).astype(jnp.float32)
        off = lax.dot(tot, triub, preferred_element_type=jnp.float32)
        rank = jnp.concatenate(
            [ranks[b] + off[:, b:b + 1] for b in range(_NABLK)], axis=1)

        counts = jnp.sum(tot, axis=1, keepdims=True)
        seg_c = jnp.floor((counts + (BLK - 1)) * (1.0 / BLK)) * BLK
        re8 = lax.broadcasted_iota(jnp.int32, (E, E), 0)
        ce8 = lax.broadcasted_iota(jnp.int32, (E, E), 1)
        tril8s = (ce8 < re8).astype(jnp.float32)
        start_c = lax.dot(tril8s, seg_c,
                          preferred_element_type=jnp.float32)
        eye8 = (re8 == ce8).astype(jnp.float32)
        start = lax.dot_general(
            start_c, eye8, (((0,), (0,)), ((), ())),
            preferred_element_type=jnp.float32)

        posT = jnp.sum((rank + start_c) * onehot, axis=0, keepdims=True)
        pos_ref[...] = posT.reshape(A // 128, 128).astype(jnp.int32)

        ptile = lax.broadcasted_iota(jnp.int32, (128, E), 0).astype(jnp.float32) * BLK
        cmp = (jnp.broadcast_to(start, (128, E)) <= ptile).astype(jnp.float32)
        eot = jnp.sum(cmp, axis=-1, keepdims=True) - 1.0

        prev = jnp.concatenate([eot[:1] - 1.0, eot[:-1]], axis=0)
        chg = (eot != prev).astype(jnp.float32)
        r128 = lax.broadcasted_iota(jnp.int32, (128, 128), 0)
        c128 = lax.broadcasted_iota(jnp.int32, (128, 128), 1)
        trili = (c128 <= r128).astype(jnp.float32)
        seg_idx = lax.dot(trili, chg, preferred_element_type=jnp.float32) - 1.0
        bslot = seg_idx - 2.0 * jnp.floor(seg_idx * 0.5)
        nxt_e = jnp.concatenate([eot[1:], eot[-1:]], axis=0)
        do_pf = jnp.concatenate([chg[1:], chg[:1] * 0.0], axis=0)
        meta = jnp.concatenate([eot, bslot, nxt_e, do_pf], axis=1)
        eot_ref[...] = meta.astype(jnp.int32)


def _router(x2d, gate_w):
    nx = T // _RBLK
    return pl.pallas_call(
        _router_body,
        grid=(nx + 1,),
        in_specs=[
            pl.BlockSpec((_RBLK, H), lambda i: (jnp.minimum(i, T // _RBLK - 1), 0)),
            pl.BlockSpec((H, E), lambda i: (0, 0)),
        ],
        out_specs=[
            pl.BlockSpec((A // 128, 128), lambda i: (0, 0)),
            pl.BlockSpec((T // 128, 128), lambda i: (0, 0)),
            pl.BlockSpec((T // 128, 128), lambda i: (0, 0)),
            pl.BlockSpec((128, 4), lambda i: (0, 0)),
        ],
        out_shape=[
            jax.ShapeDtypeStruct((A // 128, 128), jnp.int32),
            jax.ShapeDtypeStruct((T // 128, 128), jnp.float32),
            jax.ShapeDtypeStruct((T // 128, 128), jnp.float32),
            jax.ShapeDtypeStruct((128, 4), jnp.int32),
        ],
        scratch_shapes=[pltpu.VMEM((T, E), jnp.float32)],
    )(x2d, gate_w)


def _sc_dispatch_body(x_hbm, posr_hbm, xs_hbm, idx_v, rows_a, rows_b, sin, souta, soutb):
    w = lax.axis_index("s") * 2 + lax.axis_index("c")
    tbase = (w % 16) * 256
    pltpu.sync_copy(posr_hbm.at[w], idx_v)
    bufs = (rows_a, rows_b)
    outs = (souta, soutb)
    pltpu.make_async_copy(
        x_hbm.at[pl.ds(tbase, CH)], rows_a, sin).start()
    for c in range(NCH):
        cur = bufs[c % 2]
        nxt = bufs[(c + 1) % 2]
        pltpu.make_async_copy(
            x_hbm.at[pl.ds(tbase + c * CH, CH)], cur, sin).wait()
        if c + 1 < NCH:
            if c >= 1:
                pltpu.make_async_copy(
                    nxt, xs_hbm.at[idx_v.at[c - 1]], outs[(c - 1) % 2]).wait()
            pltpu.make_async_copy(
                x_hbm.at[pl.ds(tbase + (c + 1) * CH, CH)], nxt, sin).start()
        pltpu.make_async_copy(cur, xs_hbm.at[idx_v.at[c]], outs[c % 2]).start()
    pltpu.make_async_copy(
        bufs[(NCH - 2) % 2], xs_hbm.at[idx_v.at[NCH - 2]], outs[(NCH - 2) % 2]).wait()
    pltpu.make_async_copy(
        bufs[(NCH - 1) % 2], xs_hbm.at[idx_v.at[NCH - 1]], outs[(NCH - 1) % 2]).wait()


def _cast_body(w_ref, o_ref):
    o_ref[...] = w_ref[...].astype(jnp.bfloat16)


def _cast_bf16(w):
    e, m, n = w.shape
    return pl.pallas_call(
        _cast_body,
        grid=(e, 2),
        in_specs=[pl.BlockSpec((1, m // 2, n), lambda i, j: (i, j, 0))],
        out_specs=pl.BlockSpec((1, m // 2, n), lambda i, j: (i, j, 0)),
        out_shape=jax.ShapeDtypeStruct((e, m, n), jnp.bfloat16),
    )(w)


_FH = FF // 2


def _gmm_body(meta_ref, xs_ref, wg_ref, wu_ref, wd_ref, y_ref):
    xb = xs_ref[...].astype(jnp.bfloat16)
    a0 = lax.dot(xb, wg_ref[0, :, :_FH], preferred_element_type=jnp.float32)
    b0 = lax.dot(xb, wu_ref[0, :, :_FH], preferred_element_type=jnp.float32)
    h0 = (a0 * jax.nn.sigmoid(a0) * b0).astype(jnp.bfloat16)
    a1 = lax.dot(xb, wg_ref[0, :, _FH:], preferred_element_type=jnp.float32)
    b1 = lax.dot(xb, wu_ref[0, :, _FH:], preferred_element_type=jnp.float32)
    h1 = (a1 * jax.nn.sigmoid(a1) * b1).astype(jnp.bfloat16)
    y0 = lax.dot(h0, wd_ref[0, :_FH, :], preferred_element_type=jnp.float32)
    y1 = lax.dot(h1, wd_ref[0, _FH:, :], preferred_element_type=jnp.float32)
    y_ref[...] = y0 + y1


def _gmm(xs, wg_bf, wu_bf, wd_bf, meta):
    grid_spec = pltpu.PrefetchScalarGridSpec(
        num_scalar_prefetch=1,
        grid=(NT,),
        in_specs=[
            pl.BlockSpec((BLK, H), lambda i, m: (i, 0)),
            pl.BlockSpec((1, H, FF), lambda i, m: (m[i, 0], 0, 0)),
            pl.BlockSpec((1, H, FF), lambda i, m: (m[i, 0], 0, 0)),
            pl.BlockSpec((1, FF, H), lambda i, m: (m[i, 0], 0, 0)),
        ],
        out_specs=pl.BlockSpec((BLK, H), lambda i, m: (i, 0)),
    )
    return pl.pallas_call(
        _gmm_body,
        grid_spec=grid_spec,
        out_shape=jax.ShapeDtypeStruct((P, H), jnp.float32),
    )(meta, xs, wg_bf, wu_bf, wd_bf)


_CH2 = 16
_NC2 = (A // NW) // _CH2


def _sc_gather_body(y_hbm, posr2_hbm, g_hbm, idx_v, rows_a, rows_b,
                    sin_a, sin_b, souta, soutb):
    w = lax.axis_index("s") * 2 + lax.axis_index("c")
    abase = w * (A // NW)
    pltpu.sync_copy(posr2_hbm.at[w], idx_v)
    bufs = (rows_a, rows_b)
    sins = (sin_a, sin_b)
    outs = (souta, soutb)
    pltpu.make_async_copy(y_hbm.at[idx_v.at[0]], rows_a, sin_a).start()
    for c in range(_NC2):
        cur = bufs[c % 2]
        pltpu.make_async_copy(y_hbm.at[idx_v.at[c]], cur, sins[c % 2]).wait()
        if c + 1 < _NC2:
            nxt = bufs[(c + 1) % 2]
            if c >= 1:
                pltpu.make_async_copy(
                    nxt, g_hbm.at[pl.ds(abase + (c - 1) * _CH2, _CH2)],
                    outs[(c - 1) % 2]).wait()
            pltpu.make_async_copy(
                y_hbm.at[idx_v.at[c + 1]], nxt, sins[(c + 1) % 2]).start()
        pltpu.make_async_copy(
            cur, g_hbm.at[pl.ds(abase + c * _CH2, _CH2)], outs[c % 2]).start()
    pltpu.make_async_copy(
        bufs[(_NC2 - 2) % 2], g_hbm.at[pl.ds(abase + (_NC2 - 2) * _CH2, _CH2)],
        outs[(_NC2 - 2) % 2]).wait()
    pltpu.make_async_copy(
        bufs[(_NC2 - 1) % 2], g_hbm.at[pl.ds(abase + (_NC2 - 1) * _CH2, _CH2)],
        outs[(_NC2 - 1) % 2]).wait()


def _combine_body(g0_ref, g1_ref, w1_ref, w2_ref, o_ref):
    o_ref[...] = w1_ref[...] * g0_ref[...] + w2_ref[...] * g1_ref[...]


def _combine(g, w1, w2):
    return pl.pallas_call(
        _combine_body,
        grid=(T // BLK,),
        in_specs=[
            pl.BlockSpec((BLK, H), lambda i: (i, 0)),
            pl.BlockSpec((BLK, H), lambda i: (i + T // BLK, 0)),
            pl.BlockSpec((BLK, 1), lambda i: (i, 0)),
            pl.BlockSpec((BLK, 1), lambda i: (i, 0)),
        ],
        out_specs=pl.BlockSpec((BLK, H), lambda i: (i, 0)),
        out_shape=jax.ShapeDtypeStruct((T, H), jnp.float32),
    )(g, g, w1, w2)


@functools.cache
def _sc_kernels():
    mesh = plsc.VectorSubcoreMesh(
        core_axis_name="c", subcore_axis_name="s", num_cores=2, num_subcores=16)
    dispatch = pl.kernel(
        _sc_dispatch_body,
        out_type=jax.ShapeDtypeStruct((P, H), jnp.float32),
        mesh=mesh,
        scratch_types=[
            pltpu.VMEM((NCH, CH), jnp.int32),
            pltpu.VMEM((CH, H), jnp.float32),
            pltpu.VMEM((CH, H), jnp.float32),
            pltpu.SemaphoreType.DMA,
            pltpu.SemaphoreType.DMA,
            pltpu.SemaphoreType.DMA,
        ],
    )
    gather = pl.kernel(
        _sc_gather_body,
        out_type=jax.ShapeDtypeStruct((A, H), jnp.float32),
        mesh=mesh,
        scratch_types=[
            pltpu.VMEM((_NC2, _CH2), jnp.int32),
            pltpu.VMEM((_CH2, H), jnp.float32),
            pltpu.VMEM((_CH2, H), jnp.float32),
            pltpu.SemaphoreType.DMA,
            pltpu.SemaphoreType.DMA,
            pltpu.SemaphoreType.DMA,
            pltpu.SemaphoreType.DMA,
        ],
    )
    return dispatch, gather


def kernel(x, gate_w, w_gate, w_up, w_down):
    bsz, seq_len, hidden = x.shape
    x2d = x.reshape(T, H)
    pos, w1, w2, meta128 = _router(x2d, gate_w)

    pos_flat = pos.reshape(A)
    posr = pos_flat.reshape(NW, NCH, CH)
    posr2 = pos_flat.reshape(NW, _NC2, _CH2)
    meta = meta128[:NT]

    dispatch, gather = _sc_kernels()
    xs = dispatch(x2d, posr)
    wg_bf = _cast_bf16(w_gate)
    wu_bf = _cast_bf16(w_up)
    wd_bf = _cast_bf16(w_down)
    y = _gmm(xs, wg_bf, wu_bf, wd_bf, meta)
    g = gather(y, posr2)
    out = _combine(g, w1.reshape(T, 1), w2.reshape(T, 1))
    return out.reshape(bsz, seq_len, hidden)


# speedup vs baseline: 1.0092x; 1.0092x over previous
import functools

import jax
import jax.numpy as jnp
from jax import lax
from jax.experimental import pallas as pl
from jax.experimental.pallas import tpu as pltpu
from jax.experimental.pallas import tpu_sc as plsc

T = 4096
H = 2048
FF = 1408
E = 8
A = 2 * T
BLK = 256
NT = A // BLK + E
P = NT * BLK
NW = 32
CH = 16
NCH = (A // NW) // CH

_ABLK = 512
_NABLK = A // _ABLK
_RBLK = 2048


def _router_body(x_ref, gw_ref, pos_ref, w1_ref, w2_ref, eot_ref, logits):
    i = pl.program_id(0)
    nx = pl.num_programs(0) - 1

    @pl.when(i < nx)
    def _():
        xb = x_ref[...].astype(jnp.bfloat16)
        gw = gw_ref[...].astype(jnp.bfloat16)
        lgb = lax.dot(xb, gw, preferred_element_type=jnp.float32)
        logits[pl.ds(i * _RBLK, _RBLK), :] = lgb

    @pl.when(i == nx)
    def _():
        lgT = jnp.transpose(logits[...])
        row = lax.broadcasted_iota(jnp.int32, (E, T), 0)
        m1 = jnp.max(lgT, axis=0, keepdims=True)
        i1 = jnp.min(jnp.where(lgT == m1, row, E), axis=0, keepdims=True)
        lg2 = jnp.where(row == i1, -jnp.inf, lgT)
        m2 = jnp.max(lg2, axis=0, keepdims=True)
        i2 = jnp.min(jnp.where(lg2 == m2, row, E), axis=0, keepdims=True)
        s2 = jnp.exp(m2 - m1)
        w1 = 1.0 / (1.0 + s2)
        w2 = s2 / (1.0 + s2)
        w1_ref[...] = w1.reshape(T // 128, 128)
        w2_ref[...] = w2.reshape(T // 128, 128)

        row8a = lax.broadcasted_iota(jnp.int32, (E, A), 0)
        e_asn = jnp.concatenate([jnp.broadcast_to(i1, (1, T)),
                                 jnp.broadcast_to(i2, (1, T))], axis=1)
        onehot = (jnp.broadcast_to(e_asn, (E, A)) == row8a
                  ).astype(jnp.float32)

        r = lax.broadcasted_iota(jnp.int32, (_ABLK, _ABLK), 0)
        c = lax.broadcasted_iota(jnp.int32, (_ABLK, _ABLK), 1)
        triu = (r < c).astype(jnp.float32)
        ranks = []
        tots = []
        for b in range(_NABLK):
            ob = onehot[:, b * _ABLK:(b + 1) * _ABLK]
            ranks.append(lax.dot(ob, triu, preferred_element_type=jnp.float32))
            tots.append(jnp.sum(ob, axis=1, keepdims=True))
        tot = jnp.concatenate(tots, axis=1)
        rb = lax.broadcasted_iota(jnp.int32, (_NABLK, _NABLK), 0)
        cb = lax.broadcasted_iota(jnp.int32, (_NABLK, _NABLK), 1)
        triub = (rb < cb).astype(jnp.float32)
        off = lax.dot(tot, triub, preferred_element_type=jnp.float32)
        rank = jnp.concatenate(
            [ranks[b] + off[:, b:b + 1] for b in range(_NABLK)], axis=1)

        counts = jnp.sum(tot, axis=1, keepdims=True)
        seg_c = jnp.floor((counts + (BLK - 1)) * (1.0 / BLK)) * BLK
        re8 = lax.broadcasted_iota(jnp.int32, (E, E), 0)
        ce8 = lax.broadcasted_iota(jnp.int32, (E, E), 1)
        tril8s = (ce8 < re8).astype(jnp.float32)
        start_c = lax.dot(tril8s, seg_c,
                          preferred_element_type=jnp.float32)
        eye8 = (re8 == ce8).astype(jnp.float32)
        start = lax.dot_general(
            start_c, eye8, (((0,), (0,)), ((), ())),
            preferred_element_type=jnp.float32)

        posT = jnp.sum((rank + start_c) * onehot, axis=0, keepdims=True)
        pos_ref[...] = posT.reshape(A // 128, 128).astype(jnp.int32)

        ptile = lax.broadcasted_iota(jnp.int32, (128, E), 0).astype(jnp.float32) * BLK
        cmp = (jnp.broadcast_to(start, (128, E)) <= ptile).astype(jnp.float32)
        eot = jnp.sum(cmp, axis=-1, keepdims=True) - 1.0

        prev = jnp.concatenate([eot[:1] - 1.0, eot[:-1]], axis=0)
        chg = (eot != prev).astype(jnp.float32)
        r128 = lax.broadcasted_iota(jnp.int32, (128, 128), 0)
        c128 = lax.broadcasted_iota(jnp.int32, (128, 128), 1)
        trili = (c128 <= r128).astype(jnp.float32)
        seg_idx = lax.dot(trili, chg, preferred_element_type=jnp.float32) - 1.0
        bslot = seg_idx - 2.0 * jnp.floor(seg_idx * 0.5)
        nxt_e = jnp.concatenate([eot[1:], eot[-1:]], axis=0)
        do_pf = jnp.concatenate([chg[1:], chg[:1] * 0.0], axis=0)
        total = jnp.max(start + jnp.broadcast_to(
            lax.dot_general(seg_c, eye8, (((0,), (0,)), ((), ())),
                            preferred_element_type=jnp.float32), (1, E)),
            axis=-1, keepdims=True)
        active = (ptile[:, :1] < jnp.broadcast_to(total, (128, 1))
                  ).astype(jnp.float32)
        meta = jnp.concatenate([eot, bslot, nxt_e, do_pf, active], axis=1)
        eot_ref[...] = meta.astype(jnp.int32)


def _router(x2d, gate_w):
    nx = T // _RBLK
    return pl.pallas_call(
        _router_body,
        grid=(nx + 1,),
        in_specs=[
            pl.BlockSpec((_RBLK, H), lambda i: (jnp.minimum(i, T // _RBLK - 1), 0)),
            pl.BlockSpec((H, E), lambda i: (0, 0)),
        ],
        out_specs=[
            pl.BlockSpec((A // 128, 128), lambda i: (0, 0)),
            pl.BlockSpec((T // 128, 128), lambda i: (0, 0)),
            pl.BlockSpec((T // 128, 128), lambda i: (0, 0)),
            pl.BlockSpec((128, 5), lambda i: (0, 0)),
        ],
        out_shape=[
            jax.ShapeDtypeStruct((A // 128, 128), jnp.int32),
            jax.ShapeDtypeStruct((T // 128, 128), jnp.float32),
            jax.ShapeDtypeStruct((T // 128, 128), jnp.float32),
            jax.ShapeDtypeStruct((128, 5), jnp.int32),
        ],
        scratch_shapes=[pltpu.VMEM((T, E), jnp.float32)],
    )(x2d, gate_w)


def _sc_dispatch_body(x_hbm, posr_hbm, xs_hbm, idx_v, rows_a, rows_b, sin, souta, soutb):
    w = lax.axis_index("s") * 2 + lax.axis_index("c")
    tbase = (w % 16) * 256
    pltpu.sync_copy(posr_hbm.at[w], idx_v)
    bufs = (rows_a, rows_b)
    outs = (souta, soutb)
    pltpu.make_async_copy(
        x_hbm.at[pl.ds(tbase, CH)], rows_a, sin).start()
    for c in range(NCH):
        cur = bufs[c % 2]
        nxt = bufs[(c + 1) % 2]
        pltpu.make_async_copy(
            x_hbm.at[pl.ds(tbase + c * CH, CH)], cur, sin).wait()
        if c + 1 < NCH:
            if c >= 1:
                pltpu.make_async_copy(
                    nxt, xs_hbm.at[idx_v.at[c - 1]], outs[(c - 1) % 2]).wait()
            pltpu.make_async_copy(
                x_hbm.at[pl.ds(tbase + (c + 1) * CH, CH)], nxt, sin).start()
        pltpu.make_async_copy(cur, xs_hbm.at[idx_v.at[c]], outs[c % 2]).start()
    pltpu.make_async_copy(
        bufs[(NCH - 2) % 2], xs_hbm.at[idx_v.at[NCH - 2]], outs[(NCH - 2) % 2]).wait()
    pltpu.make_async_copy(
        bufs[(NCH - 1) % 2], xs_hbm.at[idx_v.at[NCH - 1]], outs[(NCH - 1) % 2]).wait()


def _cast_body(w_ref, o_ref):
    o_ref[...] = w_ref[...].astype(jnp.bfloat16)


def _cast_bf16(w):
    e, m, n = w.shape
    return pl.pallas_call(
        _cast_body,
        grid=(e, 2),
        in_specs=[pl.BlockSpec((1, m // 2, n), lambda i, j: (i, j, 0))],
        out_specs=pl.BlockSpec((1, m // 2, n), lambda i, j: (i, j, 0)),
        out_shape=jax.ShapeDtypeStruct((e, m, n), jnp.bfloat16),
    )(w)


_FH = FF // 2


def _gmm_body(meta_ref, xs_ref, wg_ref, wu_ref, wd_ref, y_ref):
    i = pl.program_id(0)

    @pl.when(meta_ref[i, 4] == 1)
    def _():
        xb = xs_ref[...].astype(jnp.bfloat16)
        a0 = lax.dot(xb, wg_ref[0, :, :_FH], preferred_element_type=jnp.float32)
        b0 = lax.dot(xb, wu_ref[0, :, :_FH], preferred_element_type=jnp.float32)
        h0 = (a0 * jax.nn.sigmoid(a0) * b0).astype(jnp.bfloat16)
        a1 = lax.dot(xb, wg_ref[0, :, _FH:], preferred_element_type=jnp.float32)
        b1 = lax.dot(xb, wu_ref[0, :, _FH:], preferred_element_type=jnp.float32)
        h1 = (a1 * jax.nn.sigmoid(a1) * b1).astype(jnp.bfloat16)
        y0 = lax.dot(h0, wd_ref[0, :_FH, :], preferred_element_type=jnp.float32)
        y1 = lax.dot(h1, wd_ref[0, _FH:, :], preferred_element_type=jnp.float32)
        y_ref[...] = y0 + y1


def _gmm(xs, wg_bf, wu_bf, wd_bf, meta):
    grid_spec = pltpu.PrefetchScalarGridSpec(
        num_scalar_prefetch=1,
        grid=(NT,),
        in_specs=[
            pl.BlockSpec((BLK, H), lambda i, m: (i, 0)),
            pl.BlockSpec((1, H, FF), lambda i, m: (m[i, 0], 0, 0)),
            pl.BlockSpec((1, H, FF), lambda i, m: (m[i, 0], 0, 0)),
            pl.BlockSpec((1, FF, H), lambda i, m: (m[i, 0], 0, 0)),
        ],
        out_specs=pl.BlockSpec((BLK, H), lambda i, m: (i, 0)),
    )
    return pl.pallas_call(
        _gmm_body,
        grid_spec=grid_spec,
        out_shape=jax.ShapeDtypeStruct((P, H), jnp.float32),
    )(meta, xs, wg_bf, wu_bf, wd_bf)


_CH2 = 16
_NC2 = (A // NW) // _CH2


def _sc_gather_body(y_hbm, posr2_hbm, g_hbm, idx_v, rows_a, rows_b,
                    sin_a, sin_b, souta, soutb):
    w = lax.axis_index("s") * 2 + lax.axis_index("c")
    abase = w * (A // NW)
    pltpu.sync_copy(posr2_hbm.at[w], idx_v)
    bufs = (rows_a, rows_b)
    sins = (sin_a, sin_b)
    outs = (souta, soutb)
    pltpu.make_async_copy(y_hbm.at[idx_v.at[0]], rows_a, sin_a).start()
    for c in range(_NC2):
        cur = bufs[c % 2]
        pltpu.make_async_copy(y_hbm.at[idx_v.at[c]], cur, sins[c % 2]).wait()
        if c + 1 < _NC2:
            nxt = bufs[(c + 1) % 2]
            if c >= 1:
                pltpu.make_async_copy(
                    nxt, g_hbm.at[pl.ds(abase + (c - 1) * _CH2, _CH2)],
                    outs[(c - 1) % 2]).wait()
            pltpu.make_async_copy(
                y_hbm.at[idx_v.at[c + 1]], nxt, sins[(c + 1) % 2]).start()
        pltpu.make_async_copy(
            cur, g_hbm.at[pl.ds(abase + c * _CH2, _CH2)], outs[c % 2]).start()
    pltpu.make_async_copy(
        bufs[(_NC2 - 2) % 2], g_hbm.at[pl.ds(abase + (_NC2 - 2) * _CH2, _CH2)],
        outs[(_NC2 - 2) % 2]).wait()
    pltpu.make_async_copy(
        bufs[(_NC2 - 1) % 2], g_hbm.at[pl.ds(abase + (_NC2 - 1) * _CH2, _CH2)],
        outs[(_NC2 - 1) % 2]).wait()


def _combine_body(g0_ref, g1_ref, w1_ref, w2_ref, o_ref):
    o_ref[...] = w1_ref[...] * g0_ref[...] + w2_ref[...] * g1_ref[...]


def _combine(g, w1, w2):
    return pl.pallas_call(
        _combine_body,
        grid=(T // BLK,),
        in_specs=[
            pl.BlockSpec((BLK, H), lambda i: (i, 0)),
            pl.BlockSpec((BLK, H), lambda i: (i + T // BLK, 0)),
            pl.BlockSpec((BLK, 1), lambda i: (i, 0)),
            pl.BlockSpec((BLK, 1), lambda i: (i, 0)),
        ],
        out_specs=pl.BlockSpec((BLK, H), lambda i: (i, 0)),
        out_shape=jax.ShapeDtypeStruct((T, H), jnp.float32),
    )(g, g, w1, w2)


@functools.cache
def _sc_kernels():
    mesh = plsc.VectorSubcoreMesh(
        core_axis_name="c", subcore_axis_name="s", num_cores=2, num_subcores=16)
    dispatch = pl.kernel(
        _sc_dispatch_body,
        out_type=jax.ShapeDtypeStruct((P, H), jnp.float32),
        mesh=mesh,
        scratch_types=[
            pltpu.VMEM((NCH, CH), jnp.int32),
            pltpu.VMEM((CH, H), jnp.float32),
            pltpu.VMEM((CH, H), jnp.float32),
            pltpu.SemaphoreType.DMA,
            pltpu.SemaphoreType.DMA,
            pltpu.SemaphoreType.DMA,
        ],
    )
    gather = pl.kernel(
        _sc_gather_body,
        out_type=jax.ShapeDtypeStruct((A, H), jnp.float32),
        mesh=mesh,
        scratch_types=[
            pltpu.VMEM((_NC2, _CH2), jnp.int32),
            pltpu.VMEM((_CH2, H), jnp.float32),
            pltpu.VMEM((_CH2, H), jnp.float32),
            pltpu.SemaphoreType.DMA,
            pltpu.SemaphoreType.DMA,
            pltpu.SemaphoreType.DMA,
            pltpu.SemaphoreType.DMA,
        ],
    )
    return dispatch, gather


def kernel(x, gate_w, w_gate, w_up, w_down):
    bsz, seq_len, hidden = x.shape
    x2d = x.reshape(T, H)
    pos, w1, w2, meta128 = _router(x2d, gate_w)

    pos_flat = pos.reshape(A)
    posr = pos_flat.reshape(NW, NCH, CH)
    posr2 = pos_flat.reshape(NW, _NC2, _CH2)
    meta = meta128[:NT]

    dispatch, gather = _sc_kernels()
    xs = dispatch(x2d, posr)
    wg_bf = _cast_bf16(w_gate)
    wu_bf = _cast_bf16(w_up)
    wd_bf = _cast_bf16(w_down)
    y = _gmm(xs, wg_bf, wu_bf, wd_bf, meta)
    g = gather(y, posr2)
    out = _combine(g, w1.reshape(T, 1), w2.reshape(T, 1))
    return out.reshape(bsz, seq_len, hidden)
